# baseline (speedup 1.0000x reference)
import jax
import jax.numpy as jnp
from jax import lax
from jax.experimental import pallas as pl
from jax.experimental.pallas import tpu as pltpu
from jax.experimental.pallas import tpu_sc as plsc

BATCH = 16384
EMBED = 64
NC = 2
NS = 16
LANES = 16
NW = NC * NS
BPW = BATCH // NW
CHUNK = 128
NCHUNK = BPW // CHUNK


def _mf_body(uid_hbm, sid_hbm, utab_hbm, stab_hbm, out_hbm,
             uid_v, sid_v, urows, srows, outv, sem_idx, sem_u, sem_s):
    wid = lax.axis_index("s") * NC + lax.axis_index("c")
    base = wid * BPW

    idx_copies = []
    for i in range(NCHUNK):
        idx_copies.append(pltpu.async_copy(
            uid_hbm.at[pl.ds(base + i * CHUNK, CHUNK)], uid_v.at[i], sem_idx))
        idx_copies.append(pltpu.async_copy(
            sid_hbm.at[pl.ds(base + i * CHUNK, CHUNK)], sid_v.at[i], sem_idx))
    for c in idx_copies:
        c.wait()

    u_copies = [pltpu.async_copy(utab_hbm.at[uid_v.at[i]],
                                 urows.at[pl.ds(i * CHUNK, CHUNK)], sem_u)
                for i in range(NCHUNK)]
    s_copies = [pltpu.async_copy(stab_hbm.at[sid_v.at[i]],
                                 srows.at[pl.ds(i * CHUNK, CHUNK)], sem_s)
                for i in range(NCHUNK)]
    for c in u_copies:
        c.wait()
    for c in s_copies:
        c.wait()

    lane = lax.iota(jnp.int32, LANES)

    def group(t, _):
        ps = []
        for r in range(LANES):
            urow = urows.at[t * LANES + r]
            srow = srows.at[t * LANES + r]
            p = None
            for c in range(EMBED // LANES):
                uv = urow[pl.ds(c * LANES, LANES)]
                sv = srow[pl.ds(c * LANES, LANES)]
                pr = uv * sv
                p = pr if p is None else p + pr
            ps.append(p)
        k = 1
        while len(ps) > 1:
            idx = jnp.bitwise_xor(lane, k)
            mask = jnp.bitwise_and(lane, k) == 0
            nxt = []
            for i in range(0, len(ps), 2):
                a, b = ps[i], ps[i + 1]
                pa = a.at[idx].get(mode="promise_in_bounds")
                pb = b.at[idx].get(mode="promise_in_bounds")
                nxt.append(jnp.where(mask, a + pa, b + pb))
            ps = nxt
            k *= 2
        dot = ps[0]
        rating = 10.0 / (1.0 + jnp.exp(-dot))
        outv[pl.ds(t * LANES, LANES)] = rating
        return _

    lax.fori_loop(0, BPW // LANES, group, None)

    pltpu.sync_copy(outv, out_hbm.at[pl.ds(base, BPW)])


def kernel(user_id, song_id, user_embedding, song_embedding):
    mesh = plsc.VectorSubcoreMesh(core_axis_name="c", subcore_axis_name="s")
    k = pl.kernel(
        _mf_body,
        mesh=mesh,
        compiler_params=pltpu.CompilerParams(
            needs_layout_passes=False, use_tc_tiling_on_sc=False),
        out_type=jax.ShapeDtypeStruct((BATCH,), jnp.float32),
        scratch_types=[
            pltpu.VMEM((NCHUNK, CHUNK), jnp.int32),
            pltpu.VMEM((NCHUNK, CHUNK), jnp.int32),
            pltpu.VMEM((BPW, EMBED), jnp.float32),
            pltpu.VMEM((BPW, EMBED), jnp.float32),
            pltpu.VMEM((BPW,), jnp.float32),
            pltpu.SemaphoreType.DMA,
            pltpu.SemaphoreType.DMA,
            pltpu.SemaphoreType.DMA,
        ],
    )
    return k(user_id.astype(jnp.int32), song_id.astype(jnp.int32),
             user_embedding, song_embedding)
